# baseline (speedup 1.0000x reference)
import functools

import jax
import jax.numpy as jnp
from jax import lax
from jax.experimental import pallas as pl
from jax.experimental.pallas import tpu as pltpu
from jax.experimental.pallas import tpu_sc as plsc

NUM_EMB = 1000000
EMBEDDING_DIM = 64
BATCH = 16384


_SC_BATCH = 8192
_INFO = plsc.get_sparse_core_info()
_NC = _INFO.num_cores
_NS = _INFO.num_subcores
_NW = _NC * _NS
_B_PER_W = _SC_BATCH // _NW
_HALF = _B_PER_W // 2
_BURST = 16
_NBURST = _HALF // _BURST
_LAG = 4

_mesh = plsc.VectorSubcoreMesh(core_axis_name="c", subcore_axis_name="s")


@functools.partial(
    pl.kernel,
    mesh=_mesh,
    compiler_params=pltpu.CompilerParams(
        needs_layout_passes=False, skip_device_barrier=True),
    cost_estimate=pl.CostEstimate(
        flops=0, transcendentals=0, bytes_accessed=8 * _SC_BATCH * 256),
    out_type=[
        jax.ShapeDtypeStruct((_SC_BATCH, EMBEDDING_DIM), jnp.float32),
        jax.ShapeDtypeStruct((_SC_BATCH, EMBEDDING_DIM), jnp.float32),
    ],
    scratch_types=[
        pltpu.VMEM((_B_PER_W,), jnp.int32),
        pltpu.VMEM((_HALF, EMBEDDING_DIM), jnp.float32),
        pltpu.VMEM((_HALF, EMBEDDING_DIM), jnp.float32),
        pltpu.SemaphoreType.DMA,
        pltpu.SemaphoreType.DMA,
    ],
)
def _sc_gather(idx_hbm, timbre_hbm, speaker_hbm, out_t_hbm, out_s_hbm,
               idx_v, rows_t, rows_s, sem_t, sem_s):
    wid = lax.axis_index("s") * _NC + lax.axis_index("c")
    base = wid * _B_PER_W
    pltpu.sync_copy(idx_hbm.at[pl.ds(base, _B_PER_W)], idx_v)
    lanes16 = lax.iota(jnp.int32, 16)

    def fire_burst(hoff, b):
        vec = idx_v[pl.ds(hoff + b * _BURST, _BURST)]
        for j in range(_BURST):
            r = jnp.sum(jnp.where(lanes16 == j, vec, 0))
            dst = b * _BURST + j
            pltpu.async_copy(timbre_hbm.at[r], rows_t.at[dst], sem_t)
            pltpu.async_copy(speaker_hbm.at[r], rows_s.at[dst], sem_s)

    def drain_burst():
        bsl = pl.ds(0, _BURST)
        pltpu.make_async_copy(timbre_hbm.at[bsl], rows_t.at[bsl], sem_t).wait()
        pltpu.make_async_copy(speaker_hbm.at[bsl], rows_s.at[bsl], sem_s).wait()

    for half in range(2):
        hoff = half * _HALF
        for b in range(_LAG):
            fire_burst(hoff, b)

        def step(b, carry):
            fire_burst(hoff, b)
            drain_burst()
            return carry

        lax.fori_loop(_LAG, _NBURST, step, 0)
        for _ in range(_LAG):
            drain_burst()
        out_sl = pl.ds(base + hoff, _HALF)
        pltpu.sync_copy(rows_t, out_t_hbm.at[out_sl])
        pltpu.sync_copy(rows_s, out_s_hbm.at[out_sl])


_TC_BATCH = BATCH - _SC_BATCH
_SEG = 1024
_NSEG = _TC_BATCH // _SEG
_UNROLL = 8
_TLAG = 32


def _tc_body(idx_ref, t_hbm, s_hbm, out_t, out_s, sem_t, sem_s):
    w = pl.program_id(0)
    base = w * _SEG

    def start(i, slot):
        r = idx_ref[base + i]
        pltpu.make_async_copy(
            t_hbm.at[pl.ds(r, 1)], out_t.at[pl.ds(slot, 1)], sem_t).start()
        pltpu.make_async_copy(
            s_hbm.at[pl.ds(r, 1)], out_s.at[pl.ds(slot, 1)], sem_s).start()

    def drain_group():
        gsl = pl.ds(0, _UNROLL)
        pltpu.make_async_copy(t_hbm.at[gsl], out_t.at[gsl], sem_t).wait()
        pltpu.make_async_copy(s_hbm.at[gsl], out_s.at[gsl], sem_s).wait()

    ngroups = _SEG // _UNROLL

    def step(g, carry):
        for k in range(_UNROLL):
            start(g * _UNROLL + k, g * _UNROLL + k)

        @pl.when(g >= _TLAG)
        def _():
            drain_group()

        return carry

    lax.fori_loop(0, ngroups, step, 0)
    for _ in range(_TLAG):
        drain_group()


def _tc_gather(idx, timbre_table, speaker_table):
    grid_spec = pltpu.PrefetchScalarGridSpec(
        num_scalar_prefetch=1,
        grid=(_NSEG,),
        in_specs=[
            pl.BlockSpec(memory_space=pl.ANY),
            pl.BlockSpec(memory_space=pl.ANY),
        ],
        out_specs=[
            pl.BlockSpec((_SEG, EMBEDDING_DIM), lambda i, idx: (i, 0)),
            pl.BlockSpec((_SEG, EMBEDDING_DIM), lambda i, idx: (i, 0)),
        ],
        scratch_shapes=[
            pltpu.SemaphoreType.DMA,
            pltpu.SemaphoreType.DMA,
        ],
    )
    return pl.pallas_call(
        _tc_body,
        grid_spec=grid_spec,
        cost_estimate=pl.CostEstimate(
            flops=0, transcendentals=0, bytes_accessed=8 * _TC_BATCH * 256),
        out_shape=[
            jax.ShapeDtypeStruct((_TC_BATCH, EMBEDDING_DIM), jnp.float32),
            jax.ShapeDtypeStruct((_TC_BATCH, EMBEDDING_DIM), jnp.float32),
        ],
    )(idx, timbre_table, speaker_table)


def kernel(inputs, timbre_table, speaker_table):
    idx = inputs.astype(jnp.int32)
    sc_t, sc_s = _sc_gather(idx[:_SC_BATCH], timbre_table, speaker_table)
    tc_t, tc_s = _tc_gather(idx[_SC_BATCH:], timbre_table, speaker_table)
    out_t = jnp.concatenate([sc_t, tc_t], axis=0)
    out_s = jnp.concatenate([sc_s, tc_s], axis=0)
    return (out_t, out_s)


# speedup vs baseline: 1.0020x; 1.0020x over previous
import functools

import jax
import jax.numpy as jnp
from jax import lax
from jax.experimental import pallas as pl
from jax.experimental.pallas import tpu as pltpu
from jax.experimental.pallas import tpu_sc as plsc

NUM_EMB = 1000000
EMBEDDING_DIM = 64
BATCH = 16384


_SC_BATCH = 8192
_INFO = plsc.get_sparse_core_info()
_NC = _INFO.num_cores
_NS = _INFO.num_subcores
_NW = _NC * _NS
_B_PER_W = _SC_BATCH // _NW
_HALF = _B_PER_W // 2
_BURST = 16
_NBURST = _HALF // _BURST
_LAG = 4

_mesh = plsc.VectorSubcoreMesh(core_axis_name="c", subcore_axis_name="s")


@functools.partial(
    pl.kernel,
    mesh=_mesh,
    compiler_params=pltpu.CompilerParams(
        needs_layout_passes=False, skip_device_barrier=True),
    cost_estimate=pl.CostEstimate(
        flops=0, transcendentals=0, bytes_accessed=8 * _SC_BATCH * 256),
    out_type=[
        jax.ShapeDtypeStruct((_SC_BATCH, EMBEDDING_DIM), jnp.float32),
        jax.ShapeDtypeStruct((_SC_BATCH, EMBEDDING_DIM), jnp.float32),
    ],
    scratch_types=[
        pltpu.VMEM((_B_PER_W,), jnp.int32),
        pltpu.VMEM((_HALF, EMBEDDING_DIM), jnp.float32),
        pltpu.VMEM((_HALF, EMBEDDING_DIM), jnp.float32),
        pltpu.SemaphoreType.DMA,
        pltpu.SemaphoreType.DMA,
    ],
)
def _sc_gather(idx_hbm, timbre_hbm, speaker_hbm, out_t_hbm, out_s_hbm,
               idx_v, rows_t, rows_s, sem_t, sem_s):
    wid = lax.axis_index("s") * _NC + lax.axis_index("c")
    base = wid * _B_PER_W
    pltpu.sync_copy(idx_hbm.at[pl.ds(base, _B_PER_W)], idx_v)
    lanes16 = lax.iota(jnp.int32, 16)

    def fire_burst(hoff, b):
        vec = idx_v[pl.ds(hoff + b * _BURST, _BURST)]
        for j in range(_BURST):
            r = jnp.sum(jnp.where(lanes16 == j, vec, 0))
            dst = b * _BURST + j
            pltpu.async_copy(timbre_hbm.at[r], rows_t.at[dst], sem_t)
            pltpu.async_copy(speaker_hbm.at[r], rows_s.at[dst], sem_s)

    def drain_burst():
        bsl = pl.ds(0, _BURST)
        pltpu.make_async_copy(timbre_hbm.at[bsl], rows_t.at[bsl], sem_t).wait()
        pltpu.make_async_copy(speaker_hbm.at[bsl], rows_s.at[bsl], sem_s).wait()

    for half in range(2):
        hoff = half * _HALF
        for b in range(_LAG):
            fire_burst(hoff, b)

        def step(b, carry):
            fire_burst(hoff, b)
            drain_burst()
            return carry

        lax.fori_loop(_LAG, _NBURST, step, 0)
        for _ in range(_LAG):
            drain_burst()
        out_sl = pl.ds(base + hoff, _HALF)
        pltpu.sync_copy(rows_t, out_t_hbm.at[out_sl])
        pltpu.sync_copy(rows_s, out_s_hbm.at[out_sl])


_TC_BATCH = BATCH - _SC_BATCH
_SEG = 1024
_NSEG = _TC_BATCH // _SEG
_UNROLL = 8
_TLAG = 32


def _tc_body(idx_ref, t_hbm, s_hbm, out_t, out_s, sem_t, sem_s):
    w = pl.program_id(0)
    base = w * _SEG

    def start(i, slot):
        r = idx_ref[_SC_BATCH + base + i]
        pltpu.make_async_copy(
            t_hbm.at[pl.ds(r, 1)], out_t.at[pl.ds(slot, 1)], sem_t).start()
        pltpu.make_async_copy(
            s_hbm.at[pl.ds(r, 1)], out_s.at[pl.ds(slot, 1)], sem_s).start()

    def drain_group():
        gsl = pl.ds(0, _UNROLL)
        pltpu.make_async_copy(t_hbm.at[gsl], out_t.at[gsl], sem_t).wait()
        pltpu.make_async_copy(s_hbm.at[gsl], out_s.at[gsl], sem_s).wait()

    ngroups = _SEG // _UNROLL

    def step(g, carry):
        for k in range(_UNROLL):
            start(g * _UNROLL + k, g * _UNROLL + k)

        @pl.when(g >= _TLAG)
        def _():
            drain_group()

        return carry

    lax.fori_loop(0, ngroups, step, 0)
    for _ in range(_TLAG):
        drain_group()


def _tc_gather(idx, timbre_table, speaker_table):
    grid_spec = pltpu.PrefetchScalarGridSpec(
        num_scalar_prefetch=1,
        grid=(_NSEG,),
        in_specs=[
            pl.BlockSpec(memory_space=pl.ANY),
            pl.BlockSpec(memory_space=pl.ANY),
        ],
        out_specs=[
            pl.BlockSpec((_SEG, EMBEDDING_DIM), lambda i, idx: (i, 0)),
            pl.BlockSpec((_SEG, EMBEDDING_DIM), lambda i, idx: (i, 0)),
        ],
        scratch_shapes=[
            pltpu.SemaphoreType.DMA,
            pltpu.SemaphoreType.DMA,
        ],
    )
    return pl.pallas_call(
        _tc_body,
        grid_spec=grid_spec,
        cost_estimate=pl.CostEstimate(
            flops=0, transcendentals=0, bytes_accessed=8 * _TC_BATCH * 256),
        out_shape=[
            jax.ShapeDtypeStruct((_TC_BATCH, EMBEDDING_DIM), jnp.float32),
            jax.ShapeDtypeStruct((_TC_BATCH, EMBEDDING_DIM), jnp.float32),
        ],
    )(idx, timbre_table, speaker_table)


def kernel(inputs, timbre_table, speaker_table):
    idx = inputs.astype(jnp.int32)
    sc_t, sc_s = _sc_gather(idx, timbre_table, speaker_table)
    tc_t, tc_s = _tc_gather(idx, timbre_table, speaker_table)
    out_t = jnp.concatenate([sc_t, tc_t], axis=0)
    out_s = jnp.concatenate([sc_s, tc_s], axis=0)
    return (out_t, out_s)


# speedup vs baseline: 1.0034x; 1.0014x over previous
import functools

import jax
import jax.numpy as jnp
from jax import lax
from jax.experimental import pallas as pl
from jax.experimental.pallas import tpu as pltpu
from jax.experimental.pallas import tpu_sc as plsc

NUM_EMB = 1000000
EMBEDDING_DIM = 64
BATCH = 16384


_SC_BATCH = 8192
_INFO = plsc.get_sparse_core_info()
_NC = _INFO.num_cores
_NS = _INFO.num_subcores
_NW = _NC * _NS
_B_PER_W = _SC_BATCH // _NW
_HALF = _B_PER_W // 2
_BURST = 16
_NBURST = _HALF // _BURST
_LAG = 4

_mesh = plsc.VectorSubcoreMesh(core_axis_name="c", subcore_axis_name="s")


@functools.partial(
    pl.kernel,
    mesh=_mesh,
    compiler_params=pltpu.CompilerParams(
        needs_layout_passes=False, skip_device_barrier=True),
    cost_estimate=pl.CostEstimate(
        flops=0, transcendentals=0, bytes_accessed=8 * _SC_BATCH * 256),
    out_type=[
        jax.ShapeDtypeStruct((_SC_BATCH, EMBEDDING_DIM), jnp.float32),
        jax.ShapeDtypeStruct((_SC_BATCH, EMBEDDING_DIM), jnp.float32),
    ],
    scratch_types=[
        pltpu.VMEM((_B_PER_W,), jnp.int32),
        pltpu.VMEM((_HALF, EMBEDDING_DIM), jnp.float32),
        pltpu.VMEM((_HALF, EMBEDDING_DIM), jnp.float32),
        pltpu.SemaphoreType.DMA,
        pltpu.SemaphoreType.DMA,
    ],
)
def _sc_gather(idx_hbm, timbre_hbm, speaker_hbm, out_t_hbm, out_s_hbm,
               idx_v, rows_t, rows_s, sem_t, sem_s):
    wid = lax.axis_index("s") * _NC + lax.axis_index("c")
    base = wid * _B_PER_W
    pltpu.sync_copy(idx_hbm.at[pl.ds(base, _B_PER_W)], idx_v)
    lanes16 = lax.iota(jnp.int32, 16)

    def fire_burst(hoff, b):
        vec = idx_v[pl.ds(hoff + b * _BURST, _BURST)]
        for j in range(_BURST):
            r = jnp.sum(jnp.where(lanes16 == j, vec, 0))
            dst = b * _BURST + j
            pltpu.async_copy(timbre_hbm.at[r], rows_t.at[dst], sem_t)
            pltpu.async_copy(speaker_hbm.at[r], rows_s.at[dst], sem_s)

    def drain_burst():
        bsl = pl.ds(0, _BURST)
        pltpu.make_async_copy(timbre_hbm.at[bsl], rows_t.at[bsl], sem_t).wait()
        pltpu.make_async_copy(speaker_hbm.at[bsl], rows_s.at[bsl], sem_s).wait()

    for half in range(2):
        hoff = half * _HALF
        for b in range(_LAG):
            fire_burst(hoff, b)

        def step(b, carry):
            fire_burst(hoff, b)
            drain_burst()
            return carry

        lax.fori_loop(_LAG, _NBURST, step, 0)
        for _ in range(_LAG):
            drain_burst()
        out_sl = pl.ds(base + hoff, _HALF)
        pltpu.sync_copy(rows_t, out_t_hbm.at[out_sl])
        pltpu.sync_copy(rows_s, out_s_hbm.at[out_sl])


_TC_BATCH = BATCH - _SC_BATCH
_SEG = 1024
_NSEG = _TC_BATCH // _SEG
_UNROLL = 8
_TLAG = 32


def _tc_body(idx_ref, t_hbm, s_hbm, out_t, out_s, sem_t, sem_s):
    w = pl.program_id(0)
    base = w * _SEG

    def start(i, slot):
        r = idx_ref[_SC_BATCH + base + i]
        pltpu.make_async_copy(
            t_hbm.at[pl.ds(r, 1)], out_t.at[pl.ds(slot, 1)], sem_t).start()
        pltpu.make_async_copy(
            s_hbm.at[pl.ds(r, 1)], out_s.at[pl.ds(slot, 1)], sem_s).start()

    def drain_group():
        gsl = pl.ds(0, _UNROLL)
        pltpu.make_async_copy(t_hbm.at[gsl], out_t.at[gsl], sem_t).wait()
        pltpu.make_async_copy(s_hbm.at[gsl], out_s.at[gsl], sem_s).wait()

    ngroups = _SEG // _UNROLL

    def step(g, carry):
        for k in range(_UNROLL):
            start(g * _UNROLL + k, g * _UNROLL + k)

        @pl.when(g >= _TLAG)
        def _():
            drain_group()

        return carry

    lax.fori_loop(0, ngroups, step, 0)
    for _ in range(_TLAG):
        drain_group()


def _tc_gather(idx, timbre_table, speaker_table):
    grid_spec = pltpu.PrefetchScalarGridSpec(
        num_scalar_prefetch=1,
        grid=(_NSEG,),
        in_specs=[
            pl.BlockSpec(memory_space=pl.ANY),
            pl.BlockSpec(memory_space=pl.ANY),
        ],
        out_specs=[
            pl.BlockSpec((_SEG, EMBEDDING_DIM), lambda i, idx: (i, 0)),
            pl.BlockSpec((_SEG, EMBEDDING_DIM), lambda i, idx: (i, 0)),
        ],
        scratch_shapes=[
            pltpu.SemaphoreType.DMA,
            pltpu.SemaphoreType.DMA,
        ],
    )
    return pl.pallas_call(
        _tc_body,
        grid_spec=grid_spec,
        compiler_params=pltpu.CompilerParams(skip_device_barrier=True),
        cost_estimate=pl.CostEstimate(
            flops=0, transcendentals=0, bytes_accessed=8 * _TC_BATCH * 256),
        out_shape=[
            jax.ShapeDtypeStruct((_TC_BATCH, EMBEDDING_DIM), jnp.float32),
            jax.ShapeDtypeStruct((_TC_BATCH, EMBEDDING_DIM), jnp.float32),
        ],
    )(idx, timbre_table, speaker_table)


def kernel(inputs, timbre_table, speaker_table):
    idx = inputs.astype(jnp.int32)
    sc_t, sc_s = _sc_gather(idx, timbre_table, speaker_table)
    tc_t, tc_s = _tc_gather(idx, timbre_table, speaker_table)
    out_t = jnp.concatenate([sc_t, tc_t], axis=0)
    out_s = jnp.concatenate([sc_s, tc_s], axis=0)
    return (out_t, out_s)


# speedup vs baseline: 1.0978x; 1.0940x over previous
import functools

import jax
import jax.numpy as jnp
from jax import lax
from jax.experimental import pallas as pl
from jax.experimental.pallas import tpu as pltpu
from jax.experimental.pallas import tpu_sc as plsc

NUM_EMB = 1000000
EMBEDDING_DIM = 64
BATCH = 16384

_INFO = plsc.get_sparse_core_info()
_NC = _INFO.num_cores
_NS = _INFO.num_subcores
_NW = _NC * _NS
_B_PER_W = BATCH // _NW
_HALF = _B_PER_W // 2
_BURST = 16
_NBURST = _HALF // _BURST
_LAG = 6

_mesh = plsc.VectorSubcoreMesh(core_axis_name="c", subcore_axis_name="s")


@functools.partial(
    pl.kernel,
    mesh=_mesh,
    compiler_params=pltpu.CompilerParams(needs_layout_passes=False),
    out_type=[
        jax.ShapeDtypeStruct((BATCH, EMBEDDING_DIM), jnp.float32),
        jax.ShapeDtypeStruct((BATCH, EMBEDDING_DIM), jnp.float32),
    ],
    scratch_types=[
        pltpu.VMEM((_B_PER_W,), jnp.int32),
        pltpu.VMEM((_HALF, EMBEDDING_DIM), jnp.float32),
        pltpu.VMEM((_HALF, EMBEDDING_DIM), jnp.float32),
        pltpu.SemaphoreType.DMA,
        pltpu.SemaphoreType.DMA,
    ],
)
def _dual_gather(idx_hbm, timbre_hbm, speaker_hbm, out_t_hbm, out_s_hbm,
                 idx_v, rows_t, rows_s, sem_t, sem_s):
    wid = lax.axis_index("s") * _NC + lax.axis_index("c")
    base = wid * _B_PER_W
    pltpu.sync_copy(idx_hbm.at[pl.ds(base, _B_PER_W)], idx_v)
    lanes16 = lax.iota(jnp.int32, 16)

    def fire_burst(hoff, b):
        vec = idx_v[pl.ds(hoff + b * _BURST, _BURST)]
        for j in range(_BURST):
            r = jnp.sum(jnp.where(lanes16 == j, vec, 0))
            dst = b * _BURST + j
            pltpu.async_copy(timbre_hbm.at[r], rows_t.at[dst], sem_t)
            pltpu.async_copy(speaker_hbm.at[r], rows_s.at[dst], sem_s)

    def drain_burst():
        bsl = pl.ds(0, _BURST)
        pltpu.make_async_copy(timbre_hbm.at[bsl], rows_t.at[bsl], sem_t).wait()
        pltpu.make_async_copy(speaker_hbm.at[bsl], rows_s.at[bsl], sem_s).wait()

    for half in range(2):
        hoff = half * _HALF
        for b in range(_LAG):
            fire_burst(hoff, b)

        def step(b, carry):
            fire_burst(hoff, b)
            drain_burst()
            return carry

        lax.fori_loop(_LAG, _NBURST, step, 0)
        for _ in range(_LAG):
            drain_burst()
        out_sl = pl.ds(base + hoff, _HALF)
        pltpu.sync_copy(rows_t, out_t_hbm.at[out_sl])
        pltpu.sync_copy(rows_s, out_s_hbm.at[out_sl])


def kernel(inputs, timbre_table, speaker_table):
    idx = inputs.astype(jnp.int32)
    out_t, out_s = _dual_gather(idx, timbre_table, speaker_table)
    return (out_t, out_s)
